# Initial kernel scaffold; baseline (speedup 1.0000x reference)
import jax
import jax.numpy as jnp
from jax.experimental import pallas as pl


def kernel(xyz):
    raise NotImplementedError("write your pallas kernel here")


# speedup vs baseline: 4.2749x; 4.2749x over previous
import jax
import jax.numpy as jnp
from jax.experimental import pallas as pl

_OFFSET = 0.001
_BLOCK_ROWS = 1024


def _recip_body(x_ref, o_ref):
    o_ref[...] = 1.0 / (jnp.abs(x_ref[...]) + _OFFSET)


def kernel(xyz):
    n, d = xyz.shape
    return pl.pallas_call(
        _recip_body,
        grid=(n // _BLOCK_ROWS,),
        in_specs=[pl.BlockSpec((_BLOCK_ROWS, d), lambda i: (i, 0))],
        out_specs=pl.BlockSpec((_BLOCK_ROWS, d), lambda i: (i, 0)),
        out_shape=jax.ShapeDtypeStruct((n, d), jnp.float32),
    )(xyz)


# speedup vs baseline: 6.7216x; 1.5724x over previous
import jax
import jax.numpy as jnp
from jax.experimental import pallas as pl

_OFFSET = 0.001
_BLOCK_ROWS = 4096


def _recip_body(x_ref, o_ref):
    o_ref[...] = 1.0 / (jnp.abs(x_ref[...]) + _OFFSET)


def kernel(xyz):
    n, d = xyz.shape
    return pl.pallas_call(
        _recip_body,
        grid=(n // _BLOCK_ROWS,),
        in_specs=[pl.BlockSpec((_BLOCK_ROWS, d), lambda i: (i, 0))],
        out_specs=pl.BlockSpec((_BLOCK_ROWS, d), lambda i: (i, 0)),
        out_shape=jax.ShapeDtypeStruct((n, d), jnp.float32),
    )(xyz)


# speedup vs baseline: 6.8264x; 1.0156x over previous
import jax
import jax.numpy as jnp
from jax.experimental import pallas as pl

_OFFSET = 0.001
_BLOCK_ROWS = 8192


def _recip_body(x_ref, o_ref):
    o_ref[...] = 1.0 / (jnp.abs(x_ref[...]) + _OFFSET)


def kernel(xyz):
    n, d = xyz.shape
    return pl.pallas_call(
        _recip_body,
        grid=(n // _BLOCK_ROWS,),
        in_specs=[pl.BlockSpec((_BLOCK_ROWS, d), lambda i: (i, 0))],
        out_specs=pl.BlockSpec((_BLOCK_ROWS, d), lambda i: (i, 0)),
        out_shape=jax.ShapeDtypeStruct((n, d), jnp.float32),
    )(xyz)
